# Initial kernel scaffold; baseline (speedup 1.0000x reference)
import jax
import jax.numpy as jnp
from jax.experimental import pallas as pl


def kernel(x, edge_index, W1_self, W1_neigh, b1, W2_self, W2_neigh, b2, W3, b3, W4, b4, W5, b5):
    raise NotImplementedError("write your pallas kernel here")


# speedup vs baseline: 4.9406x; 4.9406x over previous
import functools

import jax
import jax.numpy as jnp
from jax import lax
from jax.experimental import pallas as pl
from jax.experimental.pallas import tpu as pltpu
from jax.experimental.pallas import tpu_sc as plsc

N = 100000
E = 1600000
DW = 16
BLK = 128
NBLKP = 12512
EPAD = NBLKP * BLK - E
BLK_PER_TILE = NBLKP // 32
NP_ACC = 100096
ROWS_PER_TILE = NP_ACC // 16


@functools.cache
def _sc_kernel():
    mesh = plsc.VectorSubcoreMesh(core_axis_name="c", subcore_axis_name="s",
                                  num_cores=2, num_subcores=16)
    return functools.partial(
        pl.kernel,
        out_type=jax.ShapeDtypeStruct((2 * NP_ACC, DW), jnp.float32),
        mesh=mesh,
        scratch_types=[
            pltpu.VMEM_SHARED((NP_ACC, DW), jnp.float32),
            pltpu.VMEM((BLK,), jnp.int32),
            pltpu.VMEM((BLK,), jnp.int32),
            pltpu.VMEM((BLK, DW), jnp.float32),
            pltpu.SemaphoreType.DMA,
        ],
        compiler_params=pltpu.CompilerParams(use_tc_tiling_on_sc=False),
    )(_sc_body)


def _sc_segment_sum(tab, srcb, dstb):
    return _sc_kernel()(tab, srcb, dstb).reshape(2, NP_ACC, DW)


def _sc_body(tab, srcb, dstb, out, acc, srcv, dstv, rows, sem):
    c = lax.axis_index("c")
    s = lax.axis_index("s")
    wid = c * 16 + s

    z16 = jnp.zeros((16,), jnp.float32)

    def _zb(i, carry):
        rows[i, pl.ds(0, 16)] = z16
        return carry

    lax.fori_loop(0, BLK, _zb, 0)
    zbase = s * ROWS_PER_TILE
    for k in range(ROWS_PER_TILE // BLK):
        pltpu.sync_copy(rows, acc.at[pl.ds(zbase + k * BLK, BLK)])
    rem = ROWS_PER_TILE % BLK
    if rem:
        pltpu.sync_copy(rows.at[pl.ds(0, rem)],
                        acc.at[pl.ds(zbase + ROWS_PER_TILE - rem, rem)])
    plsc.subcore_barrier()

    start = wid * BLK_PER_TILE

    def _edge(b, carry):
        pltpu.sync_copy(srcb.at[start + b], srcv)
        pltpu.sync_copy(dstb.at[start + b], dstv)
        pltpu.async_copy(tab.at[srcv], rows, sem).wait()
        pltpu.sync_copy(rows, acc.at[dstv], add=True)
        return carry

    lax.fori_loop(0, BLK_PER_TILE, _edge, 0)
    plsc.subcore_barrier()

    pltpu.sync_copy(acc.at[pl.ds(zbase, ROWS_PER_TILE)],
                    out.at[pl.ds(c * NP_ACC + zbase, ROWS_PER_TILE)])


_R = 2000


def _combine_body(pa_ref, pb_ref, oa_ref, ob_ref):
    sa = pa_ref[0] + pa_ref[1]
    sb = pb_ref[0] + pb_ref[1]
    deg = jnp.maximum(sb[:, 1:2], 1.0)
    oa_ref[...] = sa / deg
    lane = lax.broadcasted_iota(jnp.int32, (_R, DW), 1)
    ob_ref[...] = jnp.where(lane == 0, sb / deg,
                            jnp.where(lane == 1, 1.0, 0.0))


def _combine(partsA, partsB):
    return pl.pallas_call(
        _combine_body,
        grid=(N // _R,),
        in_specs=[pl.BlockSpec((2, _R, DW), lambda i: (0, i, 0))] * 2,
        out_specs=[pl.BlockSpec((_R, DW), lambda i: (i, 0))] * 2,
        out_shape=[jax.ShapeDtypeStruct((NP_ACC, DW), jnp.float32)] * 2,
    )(partsA, partsB)


def _final_body(xt_ref, ta_ref, tb_ref, pa_ref, pb_ref, wa_ref, wb_ref,
                wc_ref, dc_ref, w3_ref, b3_ref, w4_ref, b4_ref, w5_ref,
                o_ref):
    sa = pa_ref[0] + pa_ref[1]
    sb = pb_ref[0] + pb_ref[1]
    deg = sb[:, 1:2]
    degc = jnp.maximum(deg, 1.0)
    m2a = sa / degc
    m2c = sb[:, 0:1] / degc
    x16 = xt_ref[:, 0:16]
    xc = xt_ref[:, 16:17]
    m1a = ta_ref[...]
    m1c = tb_ref[:, 0:1]
    dn = (((1,), (1,)), ((), ()))
    f32 = jnp.float32
    h2 = (lax.dot_general(x16, wa_ref[...], dn, preferred_element_type=f32)
          + lax.dot_general(m1a, wb_ref[...], dn, preferred_element_type=f32)
          + lax.dot_general(m2a, wc_ref[...], dn, preferred_element_type=f32))
    h2 = (h2 + xc * dc_ref[3:4] + m1c * dc_ref[4:5] + m2c * dc_ref[5:6])
    h2 = h2 + dc_ref[0:1] + jnp.where(deg > 0.0, 1.0, 0.0) * dc_ref[1:2]
    h3 = jnp.maximum(h2 @ w3_ref[...] + b3_ref[0:1], 0.0)
    h4 = jnp.maximum(h3 @ w4_ref[...] + b4_ref[0:1], 0.0)
    z5 = jax.nn.sigmoid(h4 @ w5_ref[...] + dc_ref[2:3, 0:1])
    o_ref[...] = z5[:, 0:1]


def _final(xt, tA, tB, pA, pB, wa, wb, wc, dc, w3, b3p, w4, b4p, w5p):
    full = lambda shape: pl.BlockSpec(shape, lambda i: tuple(0 for _ in shape))
    return pl.pallas_call(
        _final_body,
        grid=(N // _R,),
        in_specs=[
            pl.BlockSpec((_R, 17), lambda i: (i, 0)),
            pl.BlockSpec((_R, DW), lambda i: (i, 0)),
            pl.BlockSpec((_R, DW), lambda i: (i, 0)),
            pl.BlockSpec((2, _R, DW), lambda i: (0, i, 0)),
            pl.BlockSpec((2, _R, DW), lambda i: (0, i, 0)),
            full((64, 16)),
            full((64, 16)),
            full((64, 16)),
            full((8, 64)),
            full((64, 32)),
            full((8, 32)),
            full((32, 16)),
            full((8, 16)),
            full((16, 8)),
        ],
        out_specs=pl.BlockSpec((_R, 1), lambda i: (i, 0)),
        out_shape=jax.ShapeDtypeStruct((N, 1), jnp.float32),
    )(xt, tA, tB, pA, pB, wa, wb, wc, dc, w3, b3p, w4, b4p, w5p)


def kernel(x, edge_index, W1_self, W1_neigh, b1, W2_self, W2_neigh, b2,
           W3, b3, W4, b4, W5, b5):
    f32 = jnp.float32
    zpad = jnp.zeros((NP_ACC - N, DW), f32)
    tabA = jnp.concatenate([x[:, 0:16], zpad], axis=0)
    tabB = jnp.concatenate([
        jnp.concatenate([x[:, 16:17], jnp.ones((N, 1), f32),
                         jnp.zeros((N, DW - 2), f32)], axis=1), zpad], axis=0)

    pad = jnp.full((EPAD,), N, jnp.int32)
    srcp = jnp.concatenate([edge_index[0], pad]).reshape(NBLKP, BLK)
    dstp = jnp.concatenate([edge_index[1], pad]).reshape(NBLKP, BLK)

    A = (W1_self @ W2_self).T
    B = (W1_neigh @ W2_self + W1_self @ W2_neigh).T
    C = (W1_neigh @ W2_neigh).T
    d0 = b1 @ W2_self + b2
    c0 = b1 @ W2_neigh
    dc = (jnp.zeros((8, 64), f32).at[0].set(d0).at[1].set(c0)
          .at[2, 0].set(b5[0]).at[3].set(A[:, 16]).at[4].set(B[:, 16])
          .at[5].set(C[:, 16]))
    A, B, C = A[:, 0:16], B[:, 0:16], C[:, 0:16]
    b3p = jnp.zeros((8, 32), f32).at[0].set(b3)
    b4p = jnp.zeros((8, 16), f32).at[0].set(b4)
    w5p = jnp.zeros((16, 8), f32).at[:, 0].set(W5[:, 0])

    partsA1 = _sc_segment_sum(tabA, srcp, dstp)
    partsB1 = _sc_segment_sum(tabB, srcp, dstp)
    t2A, t2B = _combine(partsA1, partsB1)
    partsA2 = _sc_segment_sum(t2A, srcp, dstp)
    partsB2 = _sc_segment_sum(t2B, srcp, dstp)
    return _final(x, t2A, t2B, partsA2, partsB2,
                  A, B, C, dc, W3, b3p, W4, b4p, w5p)


# speedup vs baseline: 13.0565x; 2.6427x over previous
import functools

import jax
import jax.numpy as jnp
from jax import lax
from jax.experimental import pallas as pl
from jax.experimental.pallas import tpu as pltpu
from jax.experimental.pallas import tpu_sc as plsc

N = 100000
E = 1600000
DW = 16
BLK = 128
NBLKP = 12512
EPAD = NBLKP * BLK - E
BLK_PER_TILE = NBLKP // 32
NP_ACC = 100096
ROWS_PER_TILE = NP_ACC // 16
CH = 13
NCH = 30


@functools.cache
def _sc_kernel():
    mesh = plsc.VectorSubcoreMesh(core_axis_name="c", subcore_axis_name="s",
                                  num_cores=2, num_subcores=16)
    return functools.partial(
        pl.kernel,
        out_type=jax.ShapeDtypeStruct((2 * NP_ACC, DW), jnp.float32),
        mesh=mesh,
        scratch_types=[
            pltpu.VMEM_SHARED((NP_ACC, DW), jnp.float32),
            pltpu.VMEM((CH, BLK), jnp.int32),
            pltpu.VMEM((CH, BLK), jnp.int32),
            pltpu.VMEM((CH, BLK, DW), jnp.float32),
            pltpu.SemaphoreType.DMA,
            pltpu.SemaphoreType.DMA,
        ],
        compiler_params=pltpu.CompilerParams(use_tc_tiling_on_sc=False),
    )(_sc_body)


def _sc_segment_sum(tab, srcb, dstb):
    return _sc_kernel()(tab, srcb, dstb).reshape(2, NP_ACC, DW)


def _sc_body(tab, srcb, dstb, out, acc, sidx, didx, rows, gsem, ssem):
    c = lax.axis_index("c")
    s = lax.axis_index("s")
    wid = c * 16 + s

    z16 = jnp.zeros((16,), jnp.float32)

    def _zb(i, carry):
        rows[0, i, pl.ds(0, 16)] = z16
        return carry

    lax.fori_loop(0, BLK, _zb, 0)
    zbase = s * ROWS_PER_TILE
    for k in range(ROWS_PER_TILE // BLK):
        pltpu.sync_copy(rows.at[0], acc.at[pl.ds(zbase + k * BLK, BLK)])
    pltpu.sync_copy(rows.at[0], acc.at[pl.ds(zbase + ROWS_PER_TILE - BLK, BLK)])
    plsc.subcore_barrier()

    start = wid * BLK_PER_TILE

    def _chunk(t, carry):
        base = start + t * CH
        pltpu.sync_copy(srcb.at[pl.ds(base, CH)], sidx)
        pltpu.sync_copy(dstb.at[pl.ds(base, CH)], didx)
        gd = [pltpu.async_copy(tab.at[sidx.at[j]], rows.at[j], gsem)
              for j in range(CH)]
        for d in gd:
            d.wait()
        sd = [pltpu.async_copy(rows.at[j], acc.at[didx.at[j]], ssem, add=True)
              for j in range(CH)]
        for d in sd:
            d.wait()
        return carry

    lax.fori_loop(0, NCH, _chunk, 0)
    tbase = start + CH * NCH
    pltpu.sync_copy(srcb.at[pl.ds(tbase, 1)], sidx.at[pl.ds(0, 1)])
    pltpu.sync_copy(dstb.at[pl.ds(tbase, 1)], didx.at[pl.ds(0, 1)])
    pltpu.async_copy(tab.at[sidx.at[0]], rows.at[0], gsem).wait()
    pltpu.sync_copy(rows.at[0], acc.at[didx.at[0]], add=True)
    plsc.subcore_barrier()

    pltpu.sync_copy(acc.at[pl.ds(zbase, ROWS_PER_TILE)],
                    out.at[pl.ds(c * NP_ACC + zbase, ROWS_PER_TILE)])


_R = 2000


def _combine_body(pa_ref, pb_ref, oa_ref, ob_ref):
    sa = pa_ref[0] + pa_ref[1]
    sb = pb_ref[0] + pb_ref[1]
    deg = jnp.maximum(sb[:, 1:2], 1.0)
    oa_ref[...] = sa / deg
    lane = lax.broadcasted_iota(jnp.int32, (_R, DW), 1)
    ob_ref[...] = jnp.where(lane == 0, sb / deg,
                            jnp.where(lane == 1, 1.0, 0.0))


def _combine(partsA, partsB):
    return pl.pallas_call(
        _combine_body,
        grid=(N // _R,),
        in_specs=[pl.BlockSpec((2, _R, DW), lambda i: (0, i, 0))] * 2,
        out_specs=[pl.BlockSpec((_R, DW), lambda i: (i, 0))] * 2,
        out_shape=[jax.ShapeDtypeStruct((NP_ACC, DW), jnp.float32)] * 2,
    )(partsA, partsB)


def _final_body(xt_ref, ta_ref, tb_ref, pa_ref, pb_ref, wa_ref, wb_ref,
                wc_ref, dc_ref, w3_ref, b3_ref, w4_ref, b4_ref, w5_ref,
                o_ref):
    sa = pa_ref[0] + pa_ref[1]
    sb = pb_ref[0] + pb_ref[1]
    deg = sb[:, 1:2]
    degc = jnp.maximum(deg, 1.0)
    m2a = sa / degc
    m2c = sb[:, 0:1] / degc
    x16 = xt_ref[:, 0:16]
    xc = xt_ref[:, 16:17]
    m1a = ta_ref[...]
    m1c = tb_ref[:, 0:1]
    dn = (((1,), (1,)), ((), ()))
    f32 = jnp.float32
    h2 = (lax.dot_general(x16, wa_ref[...], dn, preferred_element_type=f32)
          + lax.dot_general(m1a, wb_ref[...], dn, preferred_element_type=f32)
          + lax.dot_general(m2a, wc_ref[...], dn, preferred_element_type=f32))
    h2 = (h2 + xc * dc_ref[3:4] + m1c * dc_ref[4:5] + m2c * dc_ref[5:6])
    h2 = h2 + dc_ref[0:1] + jnp.where(deg > 0.0, 1.0, 0.0) * dc_ref[1:2]
    h3 = jnp.maximum(h2 @ w3_ref[...] + b3_ref[0:1], 0.0)
    h4 = jnp.maximum(h3 @ w4_ref[...] + b4_ref[0:1], 0.0)
    z5 = jax.nn.sigmoid(h4 @ w5_ref[...] + dc_ref[2:3, 0:1])
    o_ref[...] = z5[:, 0:1]


def _final(xt, tA, tB, pA, pB, wa, wb, wc, dc, w3, b3p, w4, b4p, w5p):
    full = lambda shape: pl.BlockSpec(shape, lambda i: tuple(0 for _ in shape))
    return pl.pallas_call(
        _final_body,
        grid=(N // _R,),
        in_specs=[
            pl.BlockSpec((_R, 17), lambda i: (i, 0)),
            pl.BlockSpec((_R, DW), lambda i: (i, 0)),
            pl.BlockSpec((_R, DW), lambda i: (i, 0)),
            pl.BlockSpec((2, _R, DW), lambda i: (0, i, 0)),
            pl.BlockSpec((2, _R, DW), lambda i: (0, i, 0)),
            full((64, 16)),
            full((64, 16)),
            full((64, 16)),
            full((8, 64)),
            full((64, 32)),
            full((8, 32)),
            full((32, 16)),
            full((8, 16)),
            full((16, 8)),
        ],
        out_specs=pl.BlockSpec((_R, 1), lambda i: (i, 0)),
        out_shape=jax.ShapeDtypeStruct((N, 1), jnp.float32),
    )(xt, tA, tB, pA, pB, wa, wb, wc, dc, w3, b3p, w4, b4p, w5p)


def kernel(x, edge_index, W1_self, W1_neigh, b1, W2_self, W2_neigh, b2,
           W3, b3, W4, b4, W5, b5):
    f32 = jnp.float32
    zpad = jnp.zeros((NP_ACC - N, DW), f32)
    tabA = jnp.concatenate([x[:, 0:16], zpad], axis=0)
    tabB = jnp.concatenate([
        jnp.concatenate([x[:, 16:17], jnp.ones((N, 1), f32),
                         jnp.zeros((N, DW - 2), f32)], axis=1), zpad], axis=0)

    pad = jnp.full((EPAD,), N, jnp.int32)
    srcp = jnp.concatenate([edge_index[0], pad]).reshape(NBLKP, BLK)
    dstp = jnp.concatenate([edge_index[1], pad]).reshape(NBLKP, BLK)

    A = (W1_self @ W2_self).T
    B = (W1_neigh @ W2_self + W1_self @ W2_neigh).T
    C = (W1_neigh @ W2_neigh).T
    d0 = b1 @ W2_self + b2
    c0 = b1 @ W2_neigh
    dc = (jnp.zeros((8, 64), f32).at[0].set(d0).at[1].set(c0)
          .at[2, 0].set(b5[0]).at[3].set(A[:, 16]).at[4].set(B[:, 16])
          .at[5].set(C[:, 16]))
    A, B, C = A[:, 0:16], B[:, 0:16], C[:, 0:16]
    b3p = jnp.zeros((8, 32), f32).at[0].set(b3)
    b4p = jnp.zeros((8, 16), f32).at[0].set(b4)
    w5p = jnp.zeros((16, 8), f32).at[:, 0].set(W5[:, 0])

    partsA1 = _sc_segment_sum(tabA, srcp, dstp)
    partsB1 = _sc_segment_sum(tabB, srcp, dstp)
    t2A, t2B = _combine(partsA1, partsB1)
    partsA2 = _sc_segment_sum(t2A, srcp, dstp)
    partsB2 = _sc_segment_sum(t2B, srcp, dstp)
    return _final(x, t2A, t2B, partsA2, partsB2,
                  A, B, C, dc, W3, b3p, W4, b4p, w5p)


# speedup vs baseline: 18.3493x; 1.4054x over previous
import functools

import jax
import jax.numpy as jnp
from jax import lax
from jax.experimental import pallas as pl
from jax.experimental.pallas import tpu as pltpu
from jax.experimental.pallas import tpu_sc as plsc

N = 100000
E = 1600000
DW = 16
BLK = 128
NBLKP = 12512
EPAD = NBLKP * BLK - E
BLK_PER_TILE = NBLKP // 32
NP_ACC = 100096
ROWS_PER_TILE = NP_ACC // 16
CH = 13
NCH = 30


@functools.cache
def _sc_kernel():
    mesh = plsc.VectorSubcoreMesh(core_axis_name="c", subcore_axis_name="s",
                                  num_cores=2, num_subcores=16)
    return functools.partial(
        pl.kernel,
        out_type=jax.ShapeDtypeStruct((2 * NP_ACC, DW), jnp.float32),
        mesh=mesh,
        scratch_types=[
            pltpu.VMEM_SHARED((NP_ACC, DW), jnp.float32),
            pltpu.VMEM((CH, BLK), jnp.int32),
            pltpu.VMEM((CH, BLK), jnp.int32),
            pltpu.VMEM((CH, BLK, DW), jnp.float32),
            pltpu.SemaphoreType.DMA,
            pltpu.SemaphoreType.DMA,
        ],
        compiler_params=pltpu.CompilerParams(use_tc_tiling_on_sc=False),
    )(_sc_body)


def _sc_segment_sum(tab, srcb, dstb):
    return _sc_kernel()(tab, srcb, dstb).reshape(2 * NP_ACC * DW // 128, 128)


def _sc_body(tab, srcb, dstb, out, acc, sidx, didx, rows, gsem, ssem):
    c = lax.axis_index("c")
    s = lax.axis_index("s")
    wid = c * 16 + s

    z16 = jnp.zeros((16,), jnp.float32)

    def _zb(i, carry):
        rows[0, i, pl.ds(0, 16)] = z16
        return carry

    lax.fori_loop(0, BLK, _zb, 0)
    zbase = s * ROWS_PER_TILE
    for k in range(ROWS_PER_TILE // BLK):
        pltpu.sync_copy(rows.at[0], acc.at[pl.ds(zbase + k * BLK, BLK)])
    pltpu.sync_copy(rows.at[0], acc.at[pl.ds(zbase + ROWS_PER_TILE - BLK, BLK)])
    plsc.subcore_barrier()

    start = wid * BLK_PER_TILE

    def _chunk(t, carry):
        base = start + t * CH
        pltpu.sync_copy(srcb.at[pl.ds(base, CH)], sidx)
        pltpu.sync_copy(dstb.at[pl.ds(base, CH)], didx)
        gd = [pltpu.async_copy(tab.at[sidx.at[j]], rows.at[j], gsem)
              for j in range(CH)]
        for d in gd:
            d.wait()
        sd = [pltpu.async_copy(rows.at[j], acc.at[didx.at[j]], ssem, add=True)
              for j in range(CH)]
        for d in sd:
            d.wait()
        return carry

    lax.fori_loop(0, NCH, _chunk, 0)
    tbase = start + CH * NCH
    pltpu.sync_copy(srcb.at[pl.ds(tbase, 1)], sidx.at[pl.ds(0, 1)])
    pltpu.sync_copy(dstb.at[pl.ds(tbase, 1)], didx.at[pl.ds(0, 1)])
    pltpu.async_copy(tab.at[sidx.at[0]], rows.at[0], gsem).wait()
    pltpu.sync_copy(rows.at[0], acc.at[didx.at[0]], add=True)
    plsc.subcore_barrier()

    pltpu.sync_copy(acc.at[pl.ds(zbase, ROWS_PER_TILE)],
                    out.at[pl.ds(c * NP_ACC + zbase, ROWS_PER_TILE)])


MROWS = NP_ACC * DW // 128
_RC = MROWS // 4


def _combine_body(pa0_ref, pa1_ref, pb0_ref, pb1_ref, s_ref, oa_ref, ob_ref):
    sa = pa0_ref[...] + pa1_ref[...]
    sb = pb0_ref[...] + pb1_ref[...]
    degb = jnp.maximum(
        lax.dot_general(sb, s_ref[...], (((1,), (0,)), ((), ())),
                        preferred_element_type=jnp.float32), 1.0)
    oa_ref[...] = sa / degb
    lane = lax.broadcasted_iota(jnp.int32, (_RC, 128), 1) % DW
    ob_ref[...] = jnp.where(lane == 0, sb / degb,
                            jnp.where(lane == 1, 1.0, 0.0))


def _combine(partsA, partsB, S):
    return pl.pallas_call(
        _combine_body,
        grid=(4,),
        in_specs=[
            pl.BlockSpec((_RC, 128), lambda i: (i, 0)),
            pl.BlockSpec((_RC, 128), lambda i: (i + 4, 0)),
            pl.BlockSpec((_RC, 128), lambda i: (i, 0)),
            pl.BlockSpec((_RC, 128), lambda i: (i + 4, 0)),
            pl.BlockSpec((128, 128), lambda i: (0, 0)),
        ],
        out_specs=[pl.BlockSpec((_RC, 128), lambda i: (i, 0))] * 2,
        out_shape=[jax.ShapeDtypeStruct((MROWS, 128), jnp.float32)] * 2,
    )(partsA, partsA, partsB, partsB, S)


_RF = MROWS // 23


def _final_body(xv_ref, ta_ref, tb_ref, pa0_ref, pa1_ref, pb0_ref, pb1_ref,
                s_ref, abd_ref, vmat_ref, dbd_ref, w3_ref, b3_ref, w4_ref,
                b4_ref, w5_ref, b5_ref, o_ref):
    f32 = jnp.float32
    dn = (((1,), (0,)), ((), ()))
    dot = lambda a, b: lax.dot_general(a, b, dn, preferred_element_type=f32)
    sa = pa0_ref[...] + pa1_ref[...]
    sb = pb0_ref[...] + pb1_ref[...]
    degr = dot(sb, s_ref[...])
    degb = jnp.maximum(degr, 1.0)
    vin = jnp.concatenate(
        [ta_ref[...], tb_ref[...], sa / degb, sb / degb,
         jnp.where(degr > 0.0, 1.0, 0.0)], axis=1)
    h2 = dot(xv_ref[...], abd_ref[...]) + dot(vin, vmat_ref[...]) + dbd_ref[0:1]
    h3 = jnp.maximum(dot(h2, w3_ref[...]) + b3_ref[0:1], 0.0)
    h4 = jnp.maximum(dot(h3, w4_ref[...]) + b4_ref[0:1], 0.0)
    o_ref[...] = jax.nn.sigmoid(dot(h4, w5_ref[...]) + b5_ref[0:1])


def _final(xv, tA, tB, pA, pB, S, abd, vmat, dbd, w3bd, b3bd, w4bd, b4bd,
           w5bd, b5bd):
    full = lambda shape: pl.BlockSpec(shape, lambda i: tuple(0 for _ in shape))
    mspec = lambda off: pl.BlockSpec((_RF, 128), lambda i, o=off: (i + o, 0))
    return pl.pallas_call(
        _final_body,
        grid=(MROWS // _RF,),
        in_specs=[
            pl.BlockSpec((_RF, 256), lambda i: (i, 0)),
            mspec(0),
            mspec(0),
            mspec(0),
            mspec(23),
            mspec(0),
            mspec(23),
            full((128, 128)),
            full((256, 512)),
            full((640, 512)),
            full((8, 512)),
            full((512, 256)),
            full((8, 256)),
            full((256, 128)),
            full((8, 128)),
            full((128, 8)),
            full((8, 8)),
        ],
        out_specs=pl.BlockSpec((_RF, 8), lambda i: (i, 0)),
        out_shape=jax.ShapeDtypeStruct((MROWS, 8), jnp.float32),
    )(xv, tA, tB, pA, pA, pB, pB, S, abd, vmat, dbd, w3bd, b3bd, w4bd, b4bd,
      w5bd, b5bd)


def kernel(x, edge_index, W1_self, W1_neigh, b1, W2_self, W2_neigh, b2,
           W3, b3, W4, b4, W5, b5):
    f32 = jnp.float32
    zpad = jnp.zeros((NP_ACC - N, DW), f32)
    tabA = jnp.concatenate([x[:, 0:16], zpad], axis=0)
    tabB = jnp.concatenate([
        jnp.concatenate([x[:, 16:17], jnp.ones((N, 1), f32),
                         jnp.zeros((N, DW - 2), f32)], axis=1), zpad], axis=0)

    pad = jnp.full((EPAD,), N, jnp.int32)
    srcp = jnp.concatenate([edge_index[0], pad]).reshape(NBLKP, BLK)
    dstp = jnp.concatenate([edge_index[1], pad]).reshape(NBLKP, BLK)

    from jax.scipy.linalg import block_diag
    A = W1_self @ W2_self
    B = W1_neigh @ W2_self + W1_self @ W2_neigh
    C = W1_neigh @ W2_neigh
    d0 = b1 @ W2_self + b2
    c0 = b1 @ W2_neigh
    A32 = jnp.concatenate([A, jnp.zeros((15, 64), f32)], axis=0)
    row1 = lambda v: jnp.zeros((16, 64), f32).at[0].set(v)
    bd8 = lambda m: block_diag(*([m] * 8))
    abd = bd8(A32)
    vmat = jnp.concatenate(
        [bd8(B[:16]), bd8(row1(B[16])), bd8(C[:16]), bd8(row1(C[16])),
         bd8(row1(c0))], axis=0)
    rowp = lambda v, w: jnp.zeros((8, w), f32).at[0].set(jnp.tile(v, 8))
    dbd = rowp(d0, 512)
    w3bd, b3bd = bd8(W3), rowp(b3, 256)
    w4bd, b4bd = bd8(W4), rowp(b4, 128)
    w5bd, b5bd = bd8(W5), rowp(b5, 8)

    li = jnp.arange(128)
    S = ((li[:, None] % DW == 1) & (li[:, None] // DW == li[None, :] // DW)
         ).astype(f32)
    xv = jnp.concatenate(
        [x, jnp.zeros((N, 15), f32)], axis=1)
    xv = jnp.concatenate([xv, jnp.zeros((NP_ACC - N, 32), f32)], axis=0)
    xv = xv.reshape(MROWS, 256)

    partsA1 = _sc_segment_sum(tabA, srcp, dstp)
    partsB1 = _sc_segment_sum(tabB, srcp, dstp)
    t2A, t2B = _combine(partsA1, partsB1, S)
    partsA2 = _sc_segment_sum(t2A.reshape(NP_ACC, DW), srcp, dstp)
    partsB2 = _sc_segment_sum(t2B.reshape(NP_ACC, DW), srcp, dstp)
    out8 = _final(xv, t2A, t2B, partsA2, partsB2, S, abd, vmat, dbd,
                  w3bd, b3bd, w4bd, b4bd, w5bd, b5bd)
    return out8.reshape(NP_ACC, 1)[:N]
